# Initial kernel scaffold; baseline (speedup 1.0000x reference)
import jax
import jax.numpy as jnp
from jax.experimental import pallas as pl


def kernel(molnodes, moledges, ringnodes, ringedges, focused_ids, params):
    raise NotImplementedError("write your pallas kernel here")


# speedup vs baseline: 2.3695x; 2.3695x over previous
import functools

import jax
import jax.numpy as jnp
from jax import lax
from jax.experimental import pallas as pl
from jax.experimental.pallas import tpu as pltpu
from jax.experimental.pallas import tpu_sc as plsc

_B = 1024
_N = 38
_NP = 40
_NF = 45
_ET = 4
_H = 256
_GW = 128
_MP = 3
_M1O = 64
_BB = 8
_RB = 128

_SELU_ALPHA = 1.6732632423543772
_SELU_SCALE = 1.0507009873554805

_GGNN_NW = 13


def _mm(a, b, precision=lax.Precision.HIGHEST):
    return lax.dot_general(a, b, (((a.ndim - 1,), (0,)), ((), ())),
                           precision=precision,
                           preferred_element_type=jnp.float32)


def _bf(x):
    return x.astype(jnp.bfloat16)


def _rbf(x):
    return x.astype(jnp.bfloat16).astype(jnp.float32)


def _mmb(a, b):
    return lax.dot_general(_bf(a), _bf(b), (((a.ndim - 1,), (0,)), ((), ())),
                           precision=lax.Precision.DEFAULT,
                           preferred_element_type=jnp.float32)


def _selu(x):
    return _SELU_SCALE * jnp.where(x > 0, x, _SELU_ALPHA * (jnp.exp(x) - 1.0))


def _ggnn_compute(nodes3, edges3, w):
    bb = nodes3.shape[0]
    nodes = nodes3.reshape(bb * _NP, _NF)
    h = jnp.tanh(_mmb(nodes, w['W_in']) + w['b_in'])
    eb = edges3

    for _ in range(_MP):
        per = _rbf(_mmb(h, w['W_msg']) + w['b_msg'])
        per3 = per.reshape(bb, _NP, _ET * _H)
        parts = []
        for b in range(bb):
            ps = jnp.concatenate(
                [per3[b][:, e * _H:(e + 1) * _H] for e in range(_ET)], axis=0)
            parts.append(_mmb(eb[b], ps))
        msg = _rbf(jnp.concatenate(parts, axis=0))
        gx = _mmb(msg, w['Wx']) + w['bx']
        gh = _mmb(h, w['Wh']) + w['bh']
        r = jax.nn.sigmoid(gx[:, :_H] + gh[:, :_H])
        z = jax.nn.sigmoid(gx[:, _H:2 * _H] + gh[:, _H:2 * _H])
        n = jnp.tanh(gx[:, 2 * _H:] + r * gh[:, 2 * _H:])
        h = (1.0 - z) * n + z * h

    att = jax.nn.sigmoid(_mmb(h, w['W_att_h']) + _mmb(nodes, w['W_att_n']) + w['b_att'])
    emb = _mmb(h, w['W_emb']) + w['b_emb']
    prod = (att * emb).reshape(bb, _NP, _GW)
    rmask = (lax.broadcasted_iota(jnp.int32, (_NP, _GW), 0) < _N).astype(jnp.float32)
    g = jnp.sum(prod * rmask[None], axis=1)
    return g, h


_GGNN_KEYS = ('W_in', 'b_in', 'W_msg', 'b_msg', 'Wx', 'bx', 'Wh', 'bh',
              'W_att_h', 'W_att_n', 'b_att', 'W_emb', 'b_emb')


def _ring_body(nodes_ref, edges_ref, *rest):
    wrefs = rest[:_GGNN_NW]
    g_ref, h_ref = rest[_GGNN_NW], rest[_GGNN_NW + 1]
    w = {k: r[...] for k, r in zip(_GGNN_KEYS, wrefs)}
    g, h = _ggnn_compute(nodes_ref[...], edges_ref[...], w)
    g_ref[...] = g
    h_ref[...] = h.reshape(_BB, _NP, _H)


def _mol_body(nodes_ref, edges_ref, *rest):
    wrefs = rest[:_GGNN_NW]
    mrefs = rest[_GGNN_NW:_GGNN_NW + 10]
    g_ref, f_ref = rest[_GGNN_NW + 10], rest[_GGNN_NW + 11]
    w = {k: r[...] for k, r in zip(_GGNN_KEYS, wrefs)}
    g, h = _ggnn_compute(nodes_ref[...], edges_ref[...], w)
    g_ref[...] = g
    x = h
    for i in range(5):
        x = _mmb(x, mrefs[2 * i][...]) + mrefs[2 * i + 1][...]
        if i < 4:
            x = _selu(x)
    f_ref[...] = x.reshape(_BB, _NP, _M1O)[:, :_N, :]


def _const_map(nd):
    return lambda i: (0,) * nd


def _ggnn_call(nodes_p, edges_t, wlist, mlist, body, out_shape, out_specs):
    grid = (_B // _BB,)
    in_specs = [
        pl.BlockSpec((_BB, _NP, _NF), lambda i: (i, 0, 0)),
        pl.BlockSpec((_BB, _NP, _ET * _NP), lambda i: (i, 0, 0)),
    ]
    args = [nodes_p, edges_t] + wlist + mlist
    in_specs += [pl.BlockSpec(a.shape, _const_map(a.ndim)) for a in wlist + mlist]
    return pl.pallas_call(
        body,
        grid=grid,
        in_specs=in_specs,
        out_specs=out_specs,
        out_shape=out_shape,
        compiler_params=pltpu.CompilerParams(dimension_semantics=("parallel",)),
    )(*args)


def _mlp2_body(f_ref, gr_ref, gm_ref, fo_ref, *rest):
    wrefs, o_ref = rest[:-1], rest[-1]
    y = (_mmb(f_ref[...], wrefs[0][...]) + _mmb(gr_ref[...], wrefs[1][...])
         + _mmb(gm_ref[...], wrefs[2][...]) + _mmb(fo_ref[...], wrefs[3][...])
         + wrefs[4][...])
    y = _selu(y)
    for i in range(3):
        y = _selu(_mmb(y, wrefs[5 + 2 * i][...]) + wrefs[6 + 2 * i][...])
    o_ref[...] = _mmb(y, wrefs[11][...]) + wrefs[12][...]


def _mlp2_call(f1, g_ring, g_mol, focused, wlist):
    grid = (_B // _RB,)
    in_specs = [
        pl.BlockSpec((_RB, _N * _M1O), lambda i: (i, 0)),
        pl.BlockSpec((_RB, _GW), lambda i: (i, 0)),
        pl.BlockSpec((_RB, _GW), lambda i: (i, 0)),
        pl.BlockSpec((_RB, _H), lambda i: (i, 0)),
    ]
    in_specs += [pl.BlockSpec(a.shape, _const_map(a.ndim)) for a in wlist]
    dout = wlist[-1].shape[-1]
    return pl.pallas_call(
        _mlp2_body,
        grid=grid,
        in_specs=in_specs,
        out_specs=pl.BlockSpec((_RB, dout), lambda i: (i, 0)),
        out_shape=jax.ShapeDtypeStruct((_B, dout), jnp.float32),
        compiler_params=pltpu.CompilerParams(dimension_semantics=("parallel",)),
    )(f1, g_ring, g_mol, focused, *wlist)


def _sc_gather(table, idx):
    info = plsc.get_sparse_core_info()
    nw = info.num_cores * info.num_subcores
    lanes = info.num_lanes
    b_per_w = _B // nw
    mesh = plsc.VectorSubcoreMesh(core_axis_name="c", subcore_axis_name="s")

    @functools.partial(
        pl.kernel, mesh=mesh,
        out_type=jax.ShapeDtypeStruct((_B, _H), jnp.float32),
        scratch_types=[
            pltpu.VMEM((b_per_w,), jnp.int32),
            pltpu.VMEM((b_per_w, _H), jnp.float32),
            pltpu.SemaphoreType.DMA,
        ],
    )
    def k(table_hbm, idx_hbm, out_hbm, idx_v, rows_v, sem):
        wid = lax.axis_index("s") * info.num_cores + lax.axis_index("c")
        base = wid * b_per_w
        pltpu.sync_copy(idx_hbm.at[pl.ds(base, b_per_w)], idx_v)
        for c in range(b_per_w // lanes):
            sl = pl.ds(c * lanes, lanes)
            gid = lax.iota(jnp.int32, lanes) + (base + c * lanes)
            idx_v[sl] = gid * _NP + idx_v[sl]
        pltpu.async_copy(table_hbm.at[idx_v], rows_v, sem).wait()
        pltpu.sync_copy(rows_v, out_hbm.at[pl.ds(base, b_per_w)])

    return k(table, idx)


def _prep_edges(e):
    ep = jnp.pad(e, ((0, 0), (0, _NP - _N), (0, _NP - _N), (0, 0)))
    return ep.transpose(0, 1, 3, 2).reshape(_B, _NP, _ET * _NP)


def _prep_ggnn_w(p):
    return [p['W_in'], p['b_in'].reshape(1, _H),
            p['W_msg'].transpose(1, 0, 2).reshape(_H, _ET * _H),
            p['b_msg'].reshape(1, _ET * _H),
            p['Wx'], p['bx'].reshape(1, 3 * _H),
            p['Wh'], p['bh'].reshape(1, 3 * _H),
            p['W_att'][:_H], p['W_att'][_H:], p['b_att'].reshape(1, _GW),
            p['W_emb'], p['b_emb'].reshape(1, _GW)]


def kernel(molnodes, moledges, ringnodes, ringedges, focused_ids, params):
    pad_nodes = lambda n: jnp.pad(n, ((0, 0), (0, _NP - _N), (0, 0)))

    ring_w = _prep_ggnn_w(params['ring_gnn'])
    mol_w = _prep_ggnn_w(params['mol_gnn'])
    m1 = []
    for l in params['mlp1']:
        m1 += [l['W'], l['b'].reshape(1, -1)]
    p2 = params['mlp2']
    w0 = p2[0]['W']
    s0, s1, s2 = _N * _M1O, _N * _M1O + _GW, _N * _M1O + 2 * _GW
    m2 = [w0[:s0], w0[s0:s1], w0[s1:s2], w0[s2:], p2[0]['b'].reshape(1, -1)]
    for l in p2[1:]:
        m2 += [l['W'], l['b'].reshape(1, -1)]

    ring_out = _ggnn_call(
        pad_nodes(ringnodes), _prep_edges(ringedges), ring_w, [], _ring_body,
        [jax.ShapeDtypeStruct((_B, _GW), jnp.float32),
         jax.ShapeDtypeStruct((_B, _NP, _H), jnp.float32)],
        [pl.BlockSpec((_BB, _GW), lambda i: (i, 0)),
         pl.BlockSpec((_BB, _NP, _H), lambda i: (i, 0, 0))])
    g_ring, h_ring = ring_out

    mol_out = _ggnn_call(
        pad_nodes(molnodes), _prep_edges(moledges), mol_w, m1, _mol_body,
        [jax.ShapeDtypeStruct((_B, _GW), jnp.float32),
         jax.ShapeDtypeStruct((_B, _N, _M1O), jnp.float32)],
        [pl.BlockSpec((_BB, _GW), lambda i: (i, 0)),
         pl.BlockSpec((_BB, _N, _M1O), lambda i: (i, 0, 0))])
    g_mol, f1 = mol_out

    focused = _sc_gather(h_ring.reshape(_B * _NP, _H),
                         focused_ids.astype(jnp.int32))
    return _mlp2_call(f1.reshape(_B, _N * _M1O), g_ring, g_mol, focused, m2)


# speedup vs baseline: 3.0905x; 1.3043x over previous
import functools

import jax
import jax.numpy as jnp
from jax import lax
from jax.experimental import pallas as pl
from jax.experimental.pallas import tpu as pltpu
from jax.experimental.pallas import tpu_sc as plsc

_B = 1024
_N = 38
_NP = 40
_NF = 45
_ET = 4
_H = 256
_GW = 128
_MP = 3
_M1O = 64
_BB = 16
_RB = 128

_SELU_ALPHA = 1.6732632423543772
_SELU_SCALE = 1.0507009873554805

_GGNN_NW = 13


def _mm(a, b, precision=lax.Precision.HIGHEST):
    return lax.dot_general(a, b, (((a.ndim - 1,), (0,)), ((), ())),
                           precision=precision,
                           preferred_element_type=jnp.float32)


def _bf(x):
    return x.astype(jnp.bfloat16)


def _rbf(x):
    return x.astype(jnp.bfloat16).astype(jnp.float32)


def _mmb(a, b):
    return lax.dot_general(_bf(a), _bf(b), (((a.ndim - 1,), (0,)), ((), ())),
                           precision=lax.Precision.DEFAULT,
                           preferred_element_type=jnp.float32)


def _selu(x):
    return _SELU_SCALE * jnp.where(x > 0, x, _SELU_ALPHA * (jnp.exp(x) - 1.0))


def _ggnn_compute(nodes3, edges3, w):
    bb = nodes3.shape[0]
    nodes = nodes3.reshape(bb * _NP, _NF)
    h = jnp.tanh(_mmb(nodes, w['W_in']) + w['b_in'])
    eb = edges3

    for _ in range(_MP):
        per = _rbf(_mmb(h, w['W_msg']) + w['b_msg'])
        per3 = per.reshape(bb, _NP, _ET * _H)
        parts = []
        for b in range(bb):
            ps = jnp.concatenate(
                [per3[b][:, e * _H:(e + 1) * _H] for e in range(_ET)], axis=0)
            parts.append(_mmb(eb[b], ps))
        msg = _rbf(jnp.concatenate(parts, axis=0))
        gx = _mmb(msg, w['Wx']) + w['bx']
        gh = _mmb(h, w['Wh']) + w['bh']
        r = jax.nn.sigmoid(gx[:, :_H] + gh[:, :_H])
        z = jax.nn.sigmoid(gx[:, _H:2 * _H] + gh[:, _H:2 * _H])
        n = jnp.tanh(gx[:, 2 * _H:] + r * gh[:, 2 * _H:])
        h = (1.0 - z) * n + z * h

    att = jax.nn.sigmoid(_mmb(h, w['W_att_h']) + _mmb(nodes, w['W_att_n']) + w['b_att'])
    emb = _mmb(h, w['W_emb']) + w['b_emb']
    prod = (att * emb).reshape(bb, _NP, _GW)
    rmask = (lax.broadcasted_iota(jnp.int32, (_NP, _GW), 0) < _N).astype(jnp.float32)
    g = jnp.sum(prod * rmask[None], axis=1)
    return g, h


_GGNN_KEYS = ('W_in', 'b_in', 'W_msg', 'b_msg', 'Wx', 'bx', 'Wh', 'bh',
              'W_att_h', 'W_att_n', 'b_att', 'W_emb', 'b_emb')


def _ring_body(nodes_ref, edges_ref, *rest):
    wrefs = rest[:_GGNN_NW]
    g_ref, h_ref = rest[_GGNN_NW], rest[_GGNN_NW + 1]
    w = {k: r[...] for k, r in zip(_GGNN_KEYS, wrefs)}
    g, h = _ggnn_compute(nodes_ref[...], edges_ref[...], w)
    g_ref[...] = g
    h_ref[...] = h.reshape(_BB, _NP, _H)


def _mol_body(nodes_ref, edges_ref, *rest):
    wrefs = rest[:_GGNN_NW]
    mrefs = rest[_GGNN_NW:_GGNN_NW + 10]
    g_ref, f_ref = rest[_GGNN_NW + 10], rest[_GGNN_NW + 11]
    w = {k: r[...] for k, r in zip(_GGNN_KEYS, wrefs)}
    g, h = _ggnn_compute(nodes_ref[...], edges_ref[...], w)
    g_ref[...] = g
    x = h
    for i in range(5):
        x = _mmb(x, mrefs[2 * i][...]) + mrefs[2 * i + 1][...]
        if i < 4:
            x = _selu(x)
    f_ref[...] = x.reshape(_BB, _NP, _M1O)[:, :_N, :]


def _const_map(nd):
    return lambda i: (0,) * nd


def _ggnn_call(nodes_p, edges_t, wlist, mlist, body, out_shape, out_specs):
    grid = (_B // _BB,)
    in_specs = [
        pl.BlockSpec((_BB, _NP, _NF), lambda i: (i, 0, 0)),
        pl.BlockSpec((_BB, _NP, _ET * _NP), lambda i: (i, 0, 0)),
    ]
    args = [nodes_p, edges_t] + wlist + mlist
    in_specs += [pl.BlockSpec(a.shape, _const_map(a.ndim)) for a in wlist + mlist]
    return pl.pallas_call(
        body,
        grid=grid,
        in_specs=in_specs,
        out_specs=out_specs,
        out_shape=out_shape,
        compiler_params=pltpu.CompilerParams(dimension_semantics=("parallel",)),
    )(*args)


def _mlp2_body(f_ref, gr_ref, gm_ref, fo_ref, *rest):
    wrefs, o_ref = rest[:-1], rest[-1]
    y = (_mmb(f_ref[...], wrefs[0][...]) + _mmb(gr_ref[...], wrefs[1][...])
         + _mmb(gm_ref[...], wrefs[2][...]) + _mmb(fo_ref[...], wrefs[3][...])
         + wrefs[4][...])
    y = _selu(y)
    for i in range(3):
        y = _selu(_mmb(y, wrefs[5 + 2 * i][...]) + wrefs[6 + 2 * i][...])
    o_ref[...] = _mmb(y, wrefs[11][...]) + wrefs[12][...]


def _mlp2_call(f1, g_ring, g_mol, focused, wlist):
    grid = (_B // _RB,)
    in_specs = [
        pl.BlockSpec((_RB, _N * _M1O), lambda i: (i, 0)),
        pl.BlockSpec((_RB, _GW), lambda i: (i, 0)),
        pl.BlockSpec((_RB, _GW), lambda i: (i, 0)),
        pl.BlockSpec((_RB, _H), lambda i: (i, 0)),
    ]
    in_specs += [pl.BlockSpec(a.shape, _const_map(a.ndim)) for a in wlist]
    dout = wlist[-1].shape[-1]
    return pl.pallas_call(
        _mlp2_body,
        grid=grid,
        in_specs=in_specs,
        out_specs=pl.BlockSpec((_RB, dout), lambda i: (i, 0)),
        out_shape=jax.ShapeDtypeStruct((_B, dout), jnp.float32),
        compiler_params=pltpu.CompilerParams(dimension_semantics=("parallel",)),
    )(f1, g_ring, g_mol, focused, *wlist)


def _sc_gather(table, idx):
    info = plsc.get_sparse_core_info()
    nw = info.num_cores * info.num_subcores
    lanes = info.num_lanes
    b_per_w = _B // nw
    mesh = plsc.VectorSubcoreMesh(core_axis_name="c", subcore_axis_name="s")

    @functools.partial(
        pl.kernel, mesh=mesh,
        out_type=jax.ShapeDtypeStruct((_B, _H), jnp.float32),
        scratch_types=[
            pltpu.VMEM((b_per_w,), jnp.int32),
            pltpu.VMEM((b_per_w, _H), jnp.float32),
            pltpu.SemaphoreType.DMA,
        ],
    )
    def k(table_hbm, idx_hbm, out_hbm, idx_v, rows_v, sem):
        wid = lax.axis_index("s") * info.num_cores + lax.axis_index("c")
        base = wid * b_per_w
        pltpu.sync_copy(idx_hbm.at[pl.ds(base, b_per_w)], idx_v)
        for c in range(b_per_w // lanes):
            sl = pl.ds(c * lanes, lanes)
            gid = lax.iota(jnp.int32, lanes) + (base + c * lanes)
            idx_v[sl] = gid * _NP + idx_v[sl]
        pltpu.async_copy(table_hbm.at[idx_v], rows_v, sem).wait()
        pltpu.sync_copy(rows_v, out_hbm.at[pl.ds(base, b_per_w)])

    return k(table, idx)


def _prep_edges(e):
    ep = jnp.pad(e, ((0, 0), (0, _NP - _N), (0, _NP - _N), (0, 0)))
    return _bf(ep.transpose(0, 1, 3, 2).reshape(_B, _NP, _ET * _NP))


def _prep_ggnn_w(p):
    return [_bf(p['W_in']), p['b_in'].reshape(1, _H),
            _bf(p['W_msg'].transpose(1, 0, 2).reshape(_H, _ET * _H)),
            p['b_msg'].reshape(1, _ET * _H),
            _bf(p['Wx']), p['bx'].reshape(1, 3 * _H),
            _bf(p['Wh']), p['bh'].reshape(1, 3 * _H),
            _bf(p['W_att'][:_H]), _bf(p['W_att'][_H:]), p['b_att'].reshape(1, _GW),
            _bf(p['W_emb']), p['b_emb'].reshape(1, _GW)]


def kernel(molnodes, moledges, ringnodes, ringedges, focused_ids, params):
    pad_nodes = lambda n: _bf(jnp.pad(n, ((0, 0), (0, _NP - _N), (0, 0))))

    ring_w = _prep_ggnn_w(params['ring_gnn'])
    mol_w = _prep_ggnn_w(params['mol_gnn'])
    m1 = []
    for l in params['mlp1']:
        m1 += [_bf(l['W']), l['b'].reshape(1, -1)]
    p2 = params['mlp2']
    w0 = p2[0]['W']
    s0, s1, s2 = _N * _M1O, _N * _M1O + _GW, _N * _M1O + 2 * _GW
    m2 = [_bf(w0[:s0]), _bf(w0[s0:s1]), _bf(w0[s1:s2]), _bf(w0[s2:]),
          p2[0]['b'].reshape(1, -1)]
    for l in p2[1:]:
        m2 += [_bf(l['W']), l['b'].reshape(1, -1)]

    ring_out = _ggnn_call(
        pad_nodes(ringnodes), _prep_edges(ringedges), ring_w, [], _ring_body,
        [jax.ShapeDtypeStruct((_B, _GW), jnp.float32),
         jax.ShapeDtypeStruct((_B, _NP, _H), jnp.float32)],
        [pl.BlockSpec((_BB, _GW), lambda i: (i, 0)),
         pl.BlockSpec((_BB, _NP, _H), lambda i: (i, 0, 0))])
    g_ring, h_ring = ring_out

    mol_out = _ggnn_call(
        pad_nodes(molnodes), _prep_edges(moledges), mol_w, m1, _mol_body,
        [jax.ShapeDtypeStruct((_B, _GW), jnp.float32),
         jax.ShapeDtypeStruct((_B, _N, _M1O), jnp.float32)],
        [pl.BlockSpec((_BB, _GW), lambda i: (i, 0)),
         pl.BlockSpec((_BB, _N, _M1O), lambda i: (i, 0, 0))])
    g_mol, f1 = mol_out

    focused = _sc_gather(h_ring.reshape(_B * _NP, _H),
                         focused_ids.astype(jnp.int32))
    return _mlp2_call(f1.reshape(_B, _N * _M1O), g_ring, g_mol, focused, m2)


# speedup vs baseline: 3.1739x; 1.0270x over previous
import functools

import jax
import jax.numpy as jnp
from jax import lax
from jax.experimental import pallas as pl
from jax.experimental.pallas import tpu as pltpu
from jax.experimental.pallas import tpu_sc as plsc

_B = 1024
_N = 38
_NP = 40
_NF = 45
_ET = 4
_H = 256
_GW = 128
_MP = 3
_M1O = 64
_BB = 32
_RB = 256

_SELU_ALPHA = 1.6732632423543772
_SELU_SCALE = 1.0507009873554805

_GGNN_NW = 13


def _mm(a, b, precision=lax.Precision.HIGHEST):
    return lax.dot_general(a, b, (((a.ndim - 1,), (0,)), ((), ())),
                           precision=precision,
                           preferred_element_type=jnp.float32)


def _bf(x):
    return x.astype(jnp.bfloat16)


def _rbf(x):
    return x.astype(jnp.bfloat16).astype(jnp.float32)


def _mmb(a, b):
    return lax.dot_general(_bf(a), _bf(b), (((a.ndim - 1,), (0,)), ((), ())),
                           precision=lax.Precision.DEFAULT,
                           preferred_element_type=jnp.float32)


def _selu(x):
    return _SELU_SCALE * jnp.where(x > 0, x, _SELU_ALPHA * (jnp.exp(x) - 1.0))


def _ggnn_compute(nodes3, edges3, w):
    bb = nodes3.shape[0]
    nodes = nodes3.reshape(bb * _NP, _NF)
    h = jnp.tanh(_mmb(nodes, w['W_in']) + w['b_in'])
    eb = edges3

    for _ in range(_MP):
        per = _rbf(_mmb(h, w['W_msg']) + w['b_msg'])
        per3 = per.reshape(bb, _NP, _ET * _H)
        parts = []
        for b in range(bb):
            ps = jnp.concatenate(
                [per3[b][:, e * _H:(e + 1) * _H] for e in range(_ET)], axis=0)
            parts.append(_mmb(eb[b], ps))
        msg = _rbf(jnp.concatenate(parts, axis=0))
        gx = _mmb(msg, w['Wx']) + w['bx']
        gh = _mmb(h, w['Wh']) + w['bh']
        r = jax.nn.sigmoid(gx[:, :_H] + gh[:, :_H])
        z = jax.nn.sigmoid(gx[:, _H:2 * _H] + gh[:, _H:2 * _H])
        n = jnp.tanh(gx[:, 2 * _H:] + r * gh[:, 2 * _H:])
        h = (1.0 - z) * n + z * h

    att = jax.nn.sigmoid(_mmb(h, w['W_att_h']) + _mmb(nodes, w['W_att_n']) + w['b_att'])
    emb = _mmb(h, w['W_emb']) + w['b_emb']
    prod = (att * emb).reshape(bb, _NP, _GW)
    rmask = (lax.broadcasted_iota(jnp.int32, (_NP, _GW), 0) < _N).astype(jnp.float32)
    g = jnp.sum(prod * rmask[None], axis=1)
    return g, h


_GGNN_KEYS = ('W_in', 'b_in', 'W_msg', 'b_msg', 'Wx', 'bx', 'Wh', 'bh',
              'W_att_h', 'W_att_n', 'b_att', 'W_emb', 'b_emb')


def _ring_body(nodes_ref, edges_ref, *rest):
    wrefs = rest[:_GGNN_NW]
    g_ref, h_ref = rest[_GGNN_NW], rest[_GGNN_NW + 1]
    w = {k: r[...] for k, r in zip(_GGNN_KEYS, wrefs)}
    g, h = _ggnn_compute(nodes_ref[...], edges_ref[...], w)
    g_ref[...] = g
    h_ref[...] = h.reshape(_BB, _NP, _H)


def _mol_body(nodes_ref, edges_ref, *rest):
    wrefs = rest[:_GGNN_NW]
    mrefs = rest[_GGNN_NW:_GGNN_NW + 10]
    g_ref, f_ref = rest[_GGNN_NW + 10], rest[_GGNN_NW + 11]
    w = {k: r[...] for k, r in zip(_GGNN_KEYS, wrefs)}
    g, h = _ggnn_compute(nodes_ref[...], edges_ref[...], w)
    g_ref[...] = g
    x = h
    for i in range(5):
        x = _mmb(x, mrefs[2 * i][...]) + mrefs[2 * i + 1][...]
        if i < 4:
            x = _selu(x)
    f_ref[...] = x.reshape(_BB, _NP, _M1O)[:, :_N, :]


def _const_map(nd):
    return lambda i: (0,) * nd


def _ggnn_call(nodes_p, edges_t, wlist, mlist, body, out_shape, out_specs):
    grid = (_B // _BB,)
    in_specs = [
        pl.BlockSpec((_BB, _NP, _NF), lambda i: (i, 0, 0)),
        pl.BlockSpec((_BB, _NP, _ET * _NP), lambda i: (i, 0, 0)),
    ]
    args = [nodes_p, edges_t] + wlist + mlist
    in_specs += [pl.BlockSpec(a.shape, _const_map(a.ndim)) for a in wlist + mlist]
    return pl.pallas_call(
        body,
        grid=grid,
        in_specs=in_specs,
        out_specs=out_specs,
        out_shape=out_shape,
        compiler_params=pltpu.CompilerParams(dimension_semantics=("parallel",)),
    )(*args)


def _mlp2_body(f_ref, gr_ref, gm_ref, fo_ref, *rest):
    wrefs, o_ref = rest[:-1], rest[-1]
    y = (_mmb(f_ref[...], wrefs[0][...]) + _mmb(gr_ref[...], wrefs[1][...])
         + _mmb(gm_ref[...], wrefs[2][...]) + _mmb(fo_ref[...], wrefs[3][...])
         + wrefs[4][...])
    y = _selu(y)
    for i in range(3):
        y = _selu(_mmb(y, wrefs[5 + 2 * i][...]) + wrefs[6 + 2 * i][...])
    o_ref[...] = _mmb(y, wrefs[11][...]) + wrefs[12][...]


def _mlp2_call(f1, g_ring, g_mol, focused, wlist):
    grid = (_B // _RB,)
    in_specs = [
        pl.BlockSpec((_RB, _N * _M1O), lambda i: (i, 0)),
        pl.BlockSpec((_RB, _GW), lambda i: (i, 0)),
        pl.BlockSpec((_RB, _GW), lambda i: (i, 0)),
        pl.BlockSpec((_RB, _H), lambda i: (i, 0)),
    ]
    in_specs += [pl.BlockSpec(a.shape, _const_map(a.ndim)) for a in wlist]
    dout = wlist[-1].shape[-1]
    return pl.pallas_call(
        _mlp2_body,
        grid=grid,
        in_specs=in_specs,
        out_specs=pl.BlockSpec((_RB, dout), lambda i: (i, 0)),
        out_shape=jax.ShapeDtypeStruct((_B, dout), jnp.float32),
        compiler_params=pltpu.CompilerParams(dimension_semantics=("parallel",)),
    )(f1, g_ring, g_mol, focused, *wlist)


def _sc_gather(table, idx):
    info = plsc.get_sparse_core_info()
    nw = info.num_cores * info.num_subcores
    lanes = info.num_lanes
    b_per_w = _B // nw
    mesh = plsc.VectorSubcoreMesh(core_axis_name="c", subcore_axis_name="s")

    @functools.partial(
        pl.kernel, mesh=mesh,
        out_type=jax.ShapeDtypeStruct((_B, _H), jnp.float32),
        scratch_types=[
            pltpu.VMEM((b_per_w,), jnp.int32),
            pltpu.VMEM((b_per_w, _H), jnp.float32),
            pltpu.SemaphoreType.DMA,
        ],
    )
    def k(table_hbm, idx_hbm, out_hbm, idx_v, rows_v, sem):
        wid = lax.axis_index("s") * info.num_cores + lax.axis_index("c")
        base = wid * b_per_w
        pltpu.sync_copy(idx_hbm.at[pl.ds(base, b_per_w)], idx_v)
        for c in range(b_per_w // lanes):
            sl = pl.ds(c * lanes, lanes)
            gid = lax.iota(jnp.int32, lanes) + (base + c * lanes)
            idx_v[sl] = gid * _NP + idx_v[sl]
        pltpu.async_copy(table_hbm.at[idx_v], rows_v, sem).wait()
        pltpu.sync_copy(rows_v, out_hbm.at[pl.ds(base, b_per_w)])

    return k(table, idx)


def _prep_edges(e):
    ep = jnp.pad(e, ((0, 0), (0, _NP - _N), (0, _NP - _N), (0, 0)))
    return _bf(ep.transpose(0, 1, 3, 2).reshape(_B, _NP, _ET * _NP))


def _prep_ggnn_w(p):
    return [_bf(p['W_in']), p['b_in'].reshape(1, _H),
            _bf(p['W_msg'].transpose(1, 0, 2).reshape(_H, _ET * _H)),
            p['b_msg'].reshape(1, _ET * _H),
            _bf(p['Wx']), p['bx'].reshape(1, 3 * _H),
            _bf(p['Wh']), p['bh'].reshape(1, 3 * _H),
            _bf(p['W_att'][:_H]), _bf(p['W_att'][_H:]), p['b_att'].reshape(1, _GW),
            _bf(p['W_emb']), p['b_emb'].reshape(1, _GW)]


def kernel(molnodes, moledges, ringnodes, ringedges, focused_ids, params):
    pad_nodes = lambda n: _bf(jnp.pad(n, ((0, 0), (0, _NP - _N), (0, 0))))

    ring_w = _prep_ggnn_w(params['ring_gnn'])
    mol_w = _prep_ggnn_w(params['mol_gnn'])
    m1 = []
    for l in params['mlp1']:
        m1 += [_bf(l['W']), l['b'].reshape(1, -1)]
    p2 = params['mlp2']
    w0 = p2[0]['W']
    s0, s1, s2 = _N * _M1O, _N * _M1O + _GW, _N * _M1O + 2 * _GW
    m2 = [_bf(w0[:s0]), _bf(w0[s0:s1]), _bf(w0[s1:s2]), _bf(w0[s2:]),
          p2[0]['b'].reshape(1, -1)]
    for l in p2[1:]:
        m2 += [_bf(l['W']), l['b'].reshape(1, -1)]

    ring_out = _ggnn_call(
        pad_nodes(ringnodes), _prep_edges(ringedges), ring_w, [], _ring_body,
        [jax.ShapeDtypeStruct((_B, _GW), jnp.float32),
         jax.ShapeDtypeStruct((_B, _NP, _H), jnp.float32)],
        [pl.BlockSpec((_BB, _GW), lambda i: (i, 0)),
         pl.BlockSpec((_BB, _NP, _H), lambda i: (i, 0, 0))])
    g_ring, h_ring = ring_out

    mol_out = _ggnn_call(
        pad_nodes(molnodes), _prep_edges(moledges), mol_w, m1, _mol_body,
        [jax.ShapeDtypeStruct((_B, _GW), jnp.float32),
         jax.ShapeDtypeStruct((_B, _N, _M1O), jnp.float32)],
        [pl.BlockSpec((_BB, _GW), lambda i: (i, 0)),
         pl.BlockSpec((_BB, _N, _M1O), lambda i: (i, 0, 0))])
    g_mol, f1 = mol_out

    focused = _sc_gather(h_ring.reshape(_B * _NP, _H),
                         focused_ids.astype(jnp.int32))
    return _mlp2_call(f1.reshape(_B, _N * _M1O), g_ring, g_mol, focused, m2)
